# baseline (speedup 1.0000x reference)
import functools

import jax
import jax.numpy as jnp
from jax import lax
from jax.experimental import pallas as pl
from jax.experimental.pallas import tpu as pltpu
from jax.experimental.pallas import tpu_sc as plsc

_SCALE = 11.313708498984761
_D = 128
_C = 64
_NBUF = 5


@functools.partial(jax.jit, static_argnums=(0,))
def _embed(n_rows, idx, table):
    info = plsc.get_sparse_core_info()
    num_cores, num_subcores = info.num_cores, info.num_subcores
    nw = num_cores * num_subcores
    b_per_w = n_rows // nw
    g_chunks = b_per_w // _C
    g_main = (g_chunks // _NBUF) * _NBUF

    mesh = plsc.VectorSubcoreMesh(core_axis_name="c", subcore_axis_name="s")

    @functools.partial(
        pl.kernel,
        mesh=mesh,
        out_type=jax.ShapeDtypeStruct((n_rows, _D), jnp.float32),
        scratch_types=[
            pltpu.VMEM((g_chunks, _C), jnp.int32),
        ]
        + [pltpu.VMEM((_C, _D), jnp.float32) for _ in range(_NBUF)]
        + [pltpu.SemaphoreType.DMA for _ in range(2 * _NBUF)],
    )
    def k(idx_hbm, table_hbm, out_hbm, idx_v, *bufs_and_sems):
        bufs = bufs_and_sems[:_NBUF]
        gsems = bufs_and_sems[_NBUF : 2 * _NBUF]
        osems = bufs_and_sems[2 * _NBUF :]

        wid = lax.axis_index("s") * num_cores + lax.axis_index("c")
        base = wid * b_per_w
        pltpu.sync_copy(idx_hbm.at[wid], idx_v)

        def start_gather(g, b):
            pltpu.async_copy(table_hbm.at[idx_v.at[g]], bufs[b], gsems[b])

        def wait_gather(g, b):
            pltpu.make_async_copy(
                table_hbm.at[idx_v.at[g]], bufs[b], gsems[b]
            ).wait()

        def start_out(g, b):
            pltpu.async_copy(
                bufs[b], out_hbm.at[pl.ds(base + g * _C, _C)], osems[b]
            )

        def wait_out(g, b):
            pltpu.make_async_copy(
                bufs[b], out_hbm.at[pl.ds(base + g * _C, _C)], osems[b]
            ).wait()

        def scale(b):
            buf = bufs[b]

            @plsc.parallel_loop(0, _C, step=1, unroll=4)
            def _(r):
                for j in range(_D // 16):
                    sl = pl.ds(j * 16, 16)
                    buf[r, sl] = buf[r, sl] * _SCALE

        for g in range(_NBUF - 1):
            start_gather(g, g)

        def _run_if(cond):
            def deco(fn):
                if cond:
                    fn()

            return deco

        def body(g, u, dynamic):
            nb = (u + _NBUF - 1) % _NBUF

            def prefetch():
                @pl.when(g >= 1) if dynamic else _run_if(g >= 1)
                def _():
                    wait_out(g - 1, nb)

                start_gather(g + _NBUF - 1, nb)

            if dynamic:

                @pl.when(g + _NBUF - 1 < g_chunks)
                def _():
                    prefetch()

            elif g + _NBUF - 1 < g_chunks:
                prefetch()

            wait_gather(g, u)
            scale(u)
            start_out(g, u)

        def step(i, carry):
            for u in range(_NBUF):
                body(_NBUF * i + u, u, dynamic=True)
            return carry

        lax.fori_loop(0, g_main // _NBUF, step, 0)
        for g in range(g_main, g_chunks):
            body(g, g % _NBUF, dynamic=False)
        for g in range(g_chunks - _NBUF, g_chunks):
            wait_out(g, g % _NBUF)

    return k(idx, table)


def kernel(inputs, table):
    shape = inputs.shape
    n = inputs.size
    idx = inputs.reshape(32, -1, _C).astype(jnp.int32)
    out = _embed(n, idx, table)
    return out.reshape(*shape, _D)


# speedup vs baseline: 1.0034x; 1.0034x over previous
import functools

import jax
import jax.numpy as jnp
from jax import lax
from jax.experimental import pallas as pl
from jax.experimental.pallas import tpu as pltpu
from jax.experimental.pallas import tpu_sc as plsc

_SCALE = 11.313708498984761
_D = 128
_C = 128
_NBUF = 4


@functools.partial(jax.jit, static_argnums=(0,))
def _embed(n_rows, idx, table):
    info = plsc.get_sparse_core_info()
    num_cores, num_subcores = info.num_cores, info.num_subcores
    nw = num_cores * num_subcores
    b_per_w = n_rows // nw
    g_chunks = b_per_w // _C
    g_main = (g_chunks // _NBUF) * _NBUF

    mesh = plsc.VectorSubcoreMesh(core_axis_name="c", subcore_axis_name="s")

    @functools.partial(
        pl.kernel,
        mesh=mesh,
        out_type=jax.ShapeDtypeStruct((n_rows, _D), jnp.float32),
        scratch_types=[
            pltpu.VMEM((g_chunks, _C), jnp.int32),
        ]
        + [pltpu.VMEM((_C, _D), jnp.float32) for _ in range(_NBUF)]
        + [pltpu.SemaphoreType.DMA for _ in range(2 * _NBUF)],
    )
    def k(idx_hbm, table_hbm, out_hbm, idx_v, *bufs_and_sems):
        bufs = bufs_and_sems[:_NBUF]
        gsems = bufs_and_sems[_NBUF : 2 * _NBUF]
        osems = bufs_and_sems[2 * _NBUF :]

        wid = lax.axis_index("s") * num_cores + lax.axis_index("c")
        base = wid * b_per_w
        pltpu.sync_copy(idx_hbm.at[wid], idx_v)

        def start_gather(g, b):
            pltpu.async_copy(table_hbm.at[idx_v.at[g]], bufs[b], gsems[b])

        def wait_gather(g, b):
            pltpu.make_async_copy(
                table_hbm.at[idx_v.at[g]], bufs[b], gsems[b]
            ).wait()

        def start_out(g, b):
            pltpu.async_copy(
                bufs[b], out_hbm.at[pl.ds(base + g * _C, _C)], osems[b]
            )

        def wait_out(g, b):
            pltpu.make_async_copy(
                bufs[b], out_hbm.at[pl.ds(base + g * _C, _C)], osems[b]
            ).wait()

        def scale(b):
            buf = bufs[b]

            @plsc.parallel_loop(0, _C, step=1, unroll=4)
            def _(r):
                for j in range(_D // 16):
                    sl = pl.ds(j * 16, 16)
                    buf[r, sl] = buf[r, sl] * _SCALE

        for g in range(_NBUF - 1):
            start_gather(g, g)

        def _run_if(cond):
            def deco(fn):
                if cond:
                    fn()

            return deco

        def body(g, u, dynamic):
            nb = (u + _NBUF - 1) % _NBUF

            def prefetch():
                @pl.when(g >= 1) if dynamic else _run_if(g >= 1)
                def _():
                    wait_out(g - 1, nb)

                start_gather(g + _NBUF - 1, nb)

            if dynamic:

                @pl.when(g + _NBUF - 1 < g_chunks)
                def _():
                    prefetch()

            elif g + _NBUF - 1 < g_chunks:
                prefetch()

            wait_gather(g, u)
            scale(u)
            start_out(g, u)

        def step(i, carry):
            for u in range(_NBUF):
                body(_NBUF * i + u, u, dynamic=True)
            return carry

        lax.fori_loop(0, g_main // _NBUF, step, 0)
        for g in range(g_main, g_chunks):
            body(g, g % _NBUF, dynamic=False)
        for g in range(g_chunks - _NBUF, g_chunks):
            wait_out(g, g % _NBUF)

    return k(idx, table)


def kernel(inputs, table):
    shape = inputs.shape
    n = inputs.size
    idx = inputs.reshape(32, -1, _C).astype(jnp.int32)
    out = _embed(n, idx, table)
    return out.reshape(*shape, _D)


# speedup vs baseline: 1.0044x; 1.0010x over previous
import functools

import jax
import jax.numpy as jnp
from jax import lax
from jax.experimental import pallas as pl
from jax.experimental.pallas import tpu as pltpu
from jax.experimental.pallas import tpu_sc as plsc

_SCALE = 11.313708498984761
_D = 128
_C = 128
_NBUF = 4


@functools.partial(jax.jit, static_argnums=(0,))
def _embed(n_rows, idx, table):
    info = plsc.get_sparse_core_info()
    num_cores, num_subcores = info.num_cores, info.num_subcores
    nw = num_cores * num_subcores
    b_per_w = n_rows // nw
    g_chunks = b_per_w // _C
    g_main = (g_chunks // _NBUF) * _NBUF

    mesh = plsc.VectorSubcoreMesh(core_axis_name="c", subcore_axis_name="s")

    @functools.partial(
        pl.kernel,
        mesh=mesh,
        out_type=jax.ShapeDtypeStruct((n_rows, _D), jnp.float32),
        scratch_types=[
            pltpu.VMEM((g_chunks, _C), jnp.int32),
        ]
        + [pltpu.VMEM((_C, _D), jnp.float32) for _ in range(_NBUF)]
        + [pltpu.SemaphoreType.DMA for _ in range(2 * _NBUF)],
    )
    def k(idx_hbm, table_hbm, out_hbm, idx_v, *bufs_and_sems):
        bufs = bufs_and_sems[:_NBUF]
        gsems = bufs_and_sems[_NBUF : 2 * _NBUF]
        osems = bufs_and_sems[2 * _NBUF :]

        wid = lax.axis_index("s") * num_cores + lax.axis_index("c")
        base = wid * b_per_w
        pltpu.sync_copy(idx_hbm.at[wid], idx_v)

        def start_gather(g, b):
            pltpu.async_copy(table_hbm.at[idx_v.at[g]], bufs[b], gsems[b])

        def wait_gather(g, b):
            pltpu.make_async_copy(
                table_hbm.at[idx_v.at[g]], bufs[b], gsems[b]
            ).wait()

        def start_out(g, b):
            pltpu.async_copy(
                bufs[b], out_hbm.at[pl.ds(base + g * _C, _C)], osems[b]
            )

        def wait_out(g, b):
            pltpu.make_async_copy(
                bufs[b], out_hbm.at[pl.ds(base + g * _C, _C)], osems[b]
            ).wait()

        def scale(b):
            buf = bufs[b]

            @plsc.parallel_loop(0, _C, step=1, unroll=4)
            def _(r):
                for j in range(_D // 16):
                    sl = pl.ds(j * 16, 16)
                    buf[r, sl] = buf[r, sl] * _SCALE

        for g in range(_NBUF - 1):
            start_gather(g, g)

        def _run_if(cond):
            def deco(fn):
                if cond:
                    fn()

            return deco

        def body(g, u, dynamic):
            nb = (u + _NBUF - 1) % _NBUF

            def prefetch():
                @pl.when(g >= 1) if dynamic else _run_if(g >= 1)
                def _():
                    wait_out(g - 1, nb)

                start_gather(g + _NBUF - 1, nb)

            if dynamic:

                @pl.when(g + _NBUF - 1 < g_chunks)
                def _():
                    prefetch()

            elif g + _NBUF - 1 < g_chunks:
                prefetch()

            wait_gather(g, u)
            scale(u)
            start_out(g, u)

        def step(i, carry):
            for u in range(_NBUF):
                body(_NBUF * i + u, u, dynamic=True)
            return carry

        lax.fori_loop(0, g_main // _NBUF, step, 0)
        for g in range(g_main, g_chunks):
            body(g, g % _NBUF, dynamic=False)
        for g in range(g_chunks - _NBUF, g_chunks):
            wait_out(g, g % _NBUF)

    return k(idx, table)


def kernel(inputs, table):
    shape = inputs.shape
    n = inputs.size
    info = plsc.get_sparse_core_info()
    nw = info.num_cores * info.num_subcores
    idx = inputs.reshape(nw, -1, _C).astype(jnp.int32)
    out = _embed(n, idx, table)
    return out.reshape(*shape, _D)
